# baseline (speedup 1.0000x reference)
import jax
import jax.numpy as jnp
from jax.experimental import pallas as pl
from jax.experimental.pallas import tpu as pltpu
from jax.experimental.pallas import tpu_sc as plsc


_TL = 2048

_SC_ROWS = 16
_SC_LANES = 16


def _add_body(x_ref, pe_ref, o_ref):
    o_ref[...] = x_ref[...] + pe_ref[...]


def _tc_kernel(x, pos_table):
    B, L, D = x.shape
    nblk = L // _TL
    return pl.pallas_call(
        _add_body,
        grid=(nblk, B),
        in_specs=[
            pl.BlockSpec((1, _TL, D), lambda l, b: (b, l, 0)),
            pl.BlockSpec((_TL, D), lambda l, b: (l, 0)),
        ],
        out_specs=pl.BlockSpec((1, _TL, D), lambda l, b: (b, l, 0)),
        out_shape=jax.ShapeDtypeStruct((B, L, D), x.dtype),
        compiler_params=pltpu.CompilerParams(
            dimension_semantics=("parallel", "parallel"),
        ),
    )(x, pos_table)


def _sc_kernel(x, pos_table):
    B, L, D = x.shape
    x2 = x.reshape(B * L, D)
    nrow = B * L
    pe_blocks = L // _SC_ROWS

    mesh = plsc.VectorSubcoreMesh(core_axis_name="core", subcore_axis_name="subcore")

    @pl.kernel(out_type=jax.ShapeDtypeStruct((nrow, D), x.dtype), mesh=mesh,
               scratch_types=[])
    def sc_run(x_hbm, pe_hbm, o_hbm):
        unroll = 8

        def body(x_vmem, pe_vmem, o_vmem):
            @pl.loop(0, _SC_ROWS)
            def _(r):
                @pl.loop(0, D, step=_SC_LANES * unroll)
                def _(c):
                    for u in range(unroll):
                        slc = (pl.ds(r, 1), pl.ds(c + u * _SC_LANES, _SC_LANES))
                        o_vmem.at[*slc][...] = (
                            x_vmem.at[*slc][...] + pe_vmem.at[*slc][...]
                        )

        pltpu.emit_pipeline(
            body,
            grid=(nrow // _SC_ROWS,),
            in_specs=[
                pl.BlockSpec((_SC_ROWS, D), lambda i: (i, 0)),
                pl.BlockSpec((_SC_ROWS, D), lambda i: (i % pe_blocks, 0)),
            ],
            out_specs=[pl.BlockSpec((_SC_ROWS, D), lambda i: (i, 0))],
            core_axis_name=("core", "subcore"),
            dimension_semantics=(pltpu.PARALLEL,),
        )(x_hbm, pe_hbm, o_hbm)

    return sc_run(x2, pos_table).reshape(B, L, D)


def kernel(x, pos_table):
    return _sc_kernel(x, pos_table)


# speedup vs baseline: 1.0038x; 1.0038x over previous
import jax
import jax.numpy as jnp
from jax.experimental import pallas as pl
from jax.experimental.pallas import tpu as pltpu
from jax.experimental.pallas import tpu_sc as plsc


_TL = 2048

_SC_ROWS = 16
_SC_LANES = 16


def _add_body(x_ref, pe_ref, o_ref):
    o_ref[...] = x_ref[...] + pe_ref[...]


def _tc_kernel(x, pos_table):
    B, L, D = x.shape
    nblk = L // _TL
    return pl.pallas_call(
        _add_body,
        grid=(nblk, B),
        in_specs=[
            pl.BlockSpec((1, _TL, D), lambda l, b: (b, l, 0)),
            pl.BlockSpec((_TL, D), lambda l, b: (l, 0)),
        ],
        out_specs=pl.BlockSpec((1, _TL, D), lambda l, b: (b, l, 0)),
        out_shape=jax.ShapeDtypeStruct((B, L, D), x.dtype),
        compiler_params=pltpu.CompilerParams(
            dimension_semantics=("parallel", "parallel"),
        ),
    )(x, pos_table)


def _sc_kernel(x, pos_table):
    B, L, D = x.shape
    x2 = x.reshape(B * L, D)
    nrow = B * L
    pe_blocks = L // _SC_ROWS

    mesh = plsc.VectorSubcoreMesh(core_axis_name="core", subcore_axis_name="subcore")

    @pl.kernel(out_type=jax.ShapeDtypeStruct((nrow, D), x.dtype), mesh=mesh,
               scratch_types=[])
    def sc_run(x_hbm, pe_hbm, o_hbm):
        unroll = 8

        def body(x_vmem, pe_vmem, o_vmem):
            @pl.loop(0, _SC_ROWS)
            def _(r):
                @pl.loop(0, D, step=_SC_LANES * unroll)
                def _(c):
                    for u in range(unroll):
                        slc = (pl.ds(r, 1), pl.ds(c + u * _SC_LANES, _SC_LANES))
                        o_vmem.at[*slc][...] = (
                            x_vmem.at[*slc][...] + pe_vmem.at[*slc][...]
                        )

        pltpu.emit_pipeline(
            body,
            grid=(pe_blocks, B),
            in_specs=[
                pl.BlockSpec((_SC_ROWS, D), lambda i, b: (b * pe_blocks + i, 0)),
                pl.BlockSpec((_SC_ROWS, D), lambda i, b: (i, 0)),
            ],
            out_specs=[pl.BlockSpec((_SC_ROWS, D), lambda i, b: (b * pe_blocks + i, 0))],
            core_axis_name=("core", "subcore"),
            dimension_semantics=(pltpu.PARALLEL, pltpu.ARBITRARY),
        )(x_hbm, pe_hbm, o_hbm)

    return sc_run(x2, pos_table).reshape(B, L, D)


def kernel(x, pos_table):
    return _sc_kernel(x, pos_table)


# speedup vs baseline: 2.0247x; 2.0171x over previous
import jax
import jax.numpy as jnp
from jax.experimental import pallas as pl
from jax.experimental.pallas import tpu as pltpu
from jax.experimental.pallas import tpu_sc as plsc


_L_TC = 3584
_TL = 1792

_SC_ROWS = 16
_SC_LANES = 16


def _add_body(x_ref, pe_ref, o_ref):
    o_ref[...] = x_ref[...] + pe_ref[...]


def _tc_part(x, pos_table):
    B, L, D = x.shape
    nblk = _L_TC // _TL
    return pl.pallas_call(
        _add_body,
        grid=(nblk, B),
        in_specs=[
            pl.BlockSpec((1, _TL, D), lambda l, b: (b, l, 0)),
            pl.BlockSpec((_TL, D), lambda l, b: (l, 0)),
        ],
        out_specs=pl.BlockSpec((1, _TL, D), lambda l, b: (b, l, 0)),
        out_shape=jax.ShapeDtypeStruct((B, _L_TC, D), x.dtype),
        compiler_params=pltpu.CompilerParams(
            dimension_semantics=("parallel", "parallel"),
        ),
    )(x, pos_table)


def _sc_part(x, pos_table):
    B, L, D = x.shape
    l_sc = L - _L_TC
    sc_blocks = l_sc // _SC_ROWS
    pe_off = _L_TC // _SC_ROWS
    x2 = x.reshape(B * L, D)
    row_blocks_per_batch = L // _SC_ROWS

    mesh = plsc.VectorSubcoreMesh(core_axis_name="core", subcore_axis_name="subcore")

    @pl.kernel(out_type=jax.ShapeDtypeStruct((B * l_sc, D), x.dtype), mesh=mesh,
               scratch_types=[])
    def sc_run(x_hbm, pe_hbm, o_hbm):
        unroll = 8

        def body(x_vmem, pe_vmem, o_vmem):
            @pl.loop(0, _SC_ROWS)
            def _(r):
                @pl.loop(0, D, step=_SC_LANES * unroll)
                def _(c):
                    for u in range(unroll):
                        slc = (pl.ds(r, 1), pl.ds(c + u * _SC_LANES, _SC_LANES))
                        o_vmem.at[*slc][...] = (
                            x_vmem.at[*slc][...] + pe_vmem.at[*slc][...]
                        )

        pltpu.emit_pipeline(
            body,
            grid=(sc_blocks, B),
            in_specs=[
                pl.BlockSpec(
                    (_SC_ROWS, D),
                    lambda i, b: (b * row_blocks_per_batch + pe_off + i, 0),
                ),
                pl.BlockSpec((_SC_ROWS, D), lambda i, b: (pe_off + i, 0)),
            ],
            out_specs=[
                pl.BlockSpec((_SC_ROWS, D), lambda i, b: (b * sc_blocks + i, 0)),
            ],
            core_axis_name=("core", "subcore"),
            dimension_semantics=(pltpu.PARALLEL, pltpu.ARBITRARY),
        )(x_hbm, pe_hbm, o_hbm)

    return sc_run(x2, pos_table).reshape(B, l_sc, D)


def kernel(x, pos_table):
    out_tc = _tc_part(x, pos_table)
    out_sc = _sc_part(x, pos_table)
    return jnp.concatenate([out_tc, out_sc], axis=1)


# speedup vs baseline: 4.5897x; 2.2668x over previous
import jax
import jax.numpy as jnp
from jax.experimental import pallas as pl
from jax.experimental.pallas import tpu as pltpu


_TL = 2048


def _add_body(x_ref, pe_ref, o_ref):
    o_ref[...] = x_ref[...] + pe_ref[...]


def kernel(x, pos_table):
    B, L, D = x.shape
    nblk = L // _TL
    return pl.pallas_call(
        _add_body,
        grid=(nblk, B),
        in_specs=[
            pl.BlockSpec((1, _TL, D), lambda l, b: (b, l, 0)),
            pl.BlockSpec((_TL, D), lambda l, b: (l, 0)),
        ],
        out_specs=pl.BlockSpec((1, _TL, D), lambda l, b: (b, l, 0)),
        out_shape=jax.ShapeDtypeStruct((B, L, D), x.dtype),
        compiler_params=pltpu.CompilerParams(
            dimension_semantics=("parallel", "parallel"),
        ),
    )(x, pos_table)
